# Initial kernel scaffold; baseline (speedup 1.0000x reference)
import jax
import jax.numpy as jnp
from jax.experimental import pallas as pl


def kernel(features, adj, segment, W1, g1, b1, W2, g2, b2, Wf, bf, lng, lnb):
    raise NotImplementedError("write your pallas kernel here")


# speedup vs baseline: 8.6997x; 8.6997x over previous
import functools

import jax
import jax.numpy as jnp
from jax import lax
from jax.experimental import pallas as pl
from jax.experimental.pallas import tpu as pltpu
from jax.experimental.pallas import tpu_sc as plsc

NC = 2
NS = 16
NW = NC * NS
LANE = 128


def _sc_mesh():
    return plsc.VectorSubcoreMesh(
        core_axis_name="c", subcore_axis_name="s", num_cores=NC, num_subcores=NS
    )


def _sc_deg(dst2d, ones_row, zrow, NP, CPW):
    RPT = NP // NS

    @functools.partial(
        pl.kernel,
        out_type=jax.ShapeDtypeStruct((NC, NP), jnp.float32),
        mesh=_sc_mesh(),
        scratch_types=[
            pltpu.VMEM((LANE,), jnp.int32),
            pltpu.VMEM((LANE,), jnp.float32),
            pltpu.VMEM_SHARED((NP,), jnp.float32),
        ],
    )
    def k(dst_hbm, ones_hbm, z_hbm, out_hbm, dst_v, ones_v, acc):
        ci = lax.axis_index("c")
        si = lax.axis_index("s")
        wid = si * NC + ci
        pltpu.sync_copy(z_hbm, acc.at[pl.ds(si * RPT, RPT)])
        pltpu.sync_copy(ones_hbm, ones_v)
        plsc.subcore_barrier()
        base = wid * CPW

        def body(j, carry):
            pltpu.sync_copy(dst_hbm.at[base + j], dst_v)
            pltpu.sync_copy(ones_v, acc.at[dst_v], add=True)
            return carry

        lax.fori_loop(0, CPW, body, 0)
        plsc.subcore_barrier()
        pltpu.sync_copy(acc.at[pl.ds(si * RPT, RPT)],
                        out_hbm.at[ci, pl.ds(si * RPT, RPT)])

    return k(dst2d, ones_row, zrow)


def _sc_prop(y, src2d, dst2d, zrows, NP, CPW):
    RPT = NP // NS
    F = y.shape[1]

    @functools.partial(
        pl.kernel,
        out_type=jax.ShapeDtypeStruct((NC, NP, F), jnp.float32),
        mesh=_sc_mesh(),
        scratch_types=[
            pltpu.VMEM((LANE,), jnp.int32),
            pltpu.VMEM((LANE,), jnp.int32),
            pltpu.VMEM((LANE, F), jnp.float32),
            pltpu.VMEM_SHARED((NP, F), jnp.float32),
            pltpu.SemaphoreType.DMA,
        ],
    )
    def k(y_hbm, src_hbm, dst_hbm, z_hbm, out_hbm, src_v, dst_v, rows_v, acc, sem):
        ci = lax.axis_index("c")
        si = lax.axis_index("s")
        wid = si * NC + ci
        pltpu.sync_copy(z_hbm, acc.at[pl.ds(si * RPT, RPT)])
        plsc.subcore_barrier()
        base = wid * CPW

        def body(j, carry):
            pltpu.sync_copy(src_hbm.at[base + j], src_v)
            pltpu.sync_copy(dst_hbm.at[base + j], dst_v)
            pltpu.async_copy(y_hbm.at[src_v], rows_v, sem).wait()
            pltpu.sync_copy(rows_v, acc.at[dst_v], add=True)
            return carry

        lax.fori_loop(0, CPW, body, 0)
        plsc.subcore_barrier()
        pltpu.sync_copy(acc.at[pl.ds(si * RPT, RPT)],
                        out_hbm.at[ci, pl.ds(si * RPT, RPT)])

    return k(y, src2d, dst2d, zrows)


def _tc_pre(deg0, deg1, x, BR):
    NP, F = x.shape
    G = NP // BR

    def body(d0_ref, d1_ref, x_ref, u_ref, dinv_ref):
        deg = jnp.maximum(d0_ref[...] + d1_ref[...], 1.0)
        dinv = lax.rsqrt(deg)
        dinv_ref[...] = dinv
        u_ref[...] = x_ref[...] * dinv

    return pl.pallas_call(
        body,
        grid=(G,),
        in_specs=[
            pl.BlockSpec((BR, 1), lambda i: (i, 0)),
            pl.BlockSpec((BR, 1), lambda i: (i, 0)),
            pl.BlockSpec((BR, F), lambda i: (i, 0)),
        ],
        out_specs=[
            pl.BlockSpec((BR, F), lambda i: (i, 0)),
            pl.BlockSpec((BR, 1), lambda i: (i, 0)),
        ],
        out_shape=[
            jax.ShapeDtypeStruct((NP, F), jnp.float32),
            jax.ShapeDtypeStruct((NP, 1), jnp.float32),
        ],
    )(deg0, deg1, x)


def _tc_mid(pp0, pp1, dinv, BR):
    NP, F = pp0.shape
    G = NP // BR

    def body(p0_ref, p1_ref, dv_ref, tx1_ref, u_ref):
        dv = dv_ref[...]
        tx1 = -dv * (p0_ref[...] + p1_ref[...])
        tx1_ref[...] = tx1
        u_ref[...] = dv * tx1

    return pl.pallas_call(
        body,
        grid=(G,),
        in_specs=[
            pl.BlockSpec((BR, F), lambda i: (i, 0)),
            pl.BlockSpec((BR, F), lambda i: (i, 0)),
            pl.BlockSpec((BR, 1), lambda i: (i, 0)),
        ],
        out_specs=[
            pl.BlockSpec((BR, F), lambda i: (i, 0)),
            pl.BlockSpec((BR, F), lambda i: (i, 0)),
        ],
        out_shape=[
            jax.ShapeDtypeStruct((NP, F), jnp.float32),
            jax.ShapeDtypeStruct((NP, F), jnp.float32),
        ],
    )(pp0, pp1, dinv)


def _tc_layer(pp0, pp1, dinv, x_in, tx1, wcat, seg, BR):
    NP, F = x_in.shape
    C, D3, H = wcat.shape
    G = NP // BR

    def body(p0_ref, p1_ref, dv_ref, x_ref, t1_ref, w_ref, seg_ref, y_ref, s_ref):
        i = pl.program_id(0)
        dv = dv_ref[...]
        x = x_ref[...]
        t1 = t1_ref[...]
        t2 = -2.0 * dv * (p0_ref[...] + p1_ref[...]) - x
        t = jnp.concatenate([x, t1, t2], axis=1)
        seg = seg_ref[...]
        y = jnp.zeros((BR, H), jnp.float32)
        for c in range(C):
            m = jnp.where(seg == c, 1.0, 0.0)
            y = y + m * jnp.dot(t, w_ref[c], preferred_element_type=jnp.float32)
        y_ref[...] = y
        s1 = jnp.sum(y, axis=0, keepdims=True)
        s2 = jnp.sum(y * y, axis=0, keepdims=True)
        rows = lax.broadcasted_iota(jnp.int32, (8, H), 0)
        sblk = (jnp.where(rows == 0, jnp.broadcast_to(s1, (8, H)), 0.0)
                + jnp.where(rows == 1, jnp.broadcast_to(s2, (8, H)), 0.0))

        @pl.when(i == 0)
        def _():
            s_ref[...] = sblk

        @pl.when(i > 0)
        def _():
            s_ref[...] = s_ref[...] + sblk

    return pl.pallas_call(
        body,
        grid=(G,),
        in_specs=[
            pl.BlockSpec((BR, F), lambda i: (i, 0)),
            pl.BlockSpec((BR, F), lambda i: (i, 0)),
            pl.BlockSpec((BR, 1), lambda i: (i, 0)),
            pl.BlockSpec((BR, F), lambda i: (i, 0)),
            pl.BlockSpec((BR, F), lambda i: (i, 0)),
            pl.BlockSpec((C, D3, H), lambda i: (0, 0, 0)),
            pl.BlockSpec((BR, 1), lambda i: (i, 0)),
        ],
        out_specs=[
            pl.BlockSpec((BR, H), lambda i: (i, 0)),
            pl.BlockSpec((8, H), lambda i: (0, 0)),
        ],
        out_shape=[
            jax.ShapeDtypeStruct((NP, H), jnp.float32),
            jax.ShapeDtypeStruct((8, H), jnp.float32),
        ],
    )(pp0, pp1, dinv, x_in, tx1, wcat, seg)


def _tc_post1(y, sums, g, b, dinv, n_real, BR):
    NP, H = y.shape
    G = NP // BR

    def body(y_ref, s_ref, g_ref, b_ref, dv_ref, x_ref, u_ref):
        i = pl.program_id(0)
        s1 = s_ref[0:1, :]
        s2 = s_ref[1:2, :]
        mu = s1 / n_real
        var = s2 / n_real - mu * mu
        inv = lax.rsqrt(var + 1e-5)
        yb = g_ref[...] * (y_ref[...] - mu) * inv + b_ref[...]
        xn = yb * (1.0 / (1.0 + jnp.exp(-yb)))
        rows = i * BR + lax.broadcasted_iota(jnp.int32, (BR, 1), 0)
        xn = jnp.where(rows < n_real, xn, 0.0)
        x_ref[...] = xn
        u_ref[...] = dv_ref[...] * xn

    return pl.pallas_call(
        body,
        grid=(G,),
        in_specs=[
            pl.BlockSpec((BR, H), lambda i: (i, 0)),
            pl.BlockSpec((8, H), lambda i: (0, 0)),
            pl.BlockSpec((1, H), lambda i: (0, 0)),
            pl.BlockSpec((1, H), lambda i: (0, 0)),
            pl.BlockSpec((BR, 1), lambda i: (i, 0)),
        ],
        out_specs=[
            pl.BlockSpec((BR, H), lambda i: (i, 0)),
            pl.BlockSpec((BR, H), lambda i: (i, 0)),
        ],
        out_shape=[
            jax.ShapeDtypeStruct((NP, H), jnp.float32),
            jax.ShapeDtypeStruct((NP, H), jnp.float32),
        ],
    )(y, sums, g, b, dinv)


def _tc_post2(y, sums, g, b, wfc, bfc, seg, n_real, BR):
    NP, H = y.shape
    C = wfc.shape[0]
    G = NP // BR

    def body(y_ref, s_ref, g_ref, b_ref, wf_ref, bf_ref, seg_ref, v_ref, vs_ref):
        i = pl.program_id(0)
        s1 = s_ref[0:1, :]
        s2 = s_ref[1:2, :]
        mu = s1 / n_real
        var = s2 / n_real - mu * mu
        inv = lax.rsqrt(var + 1e-5)
        yb = g_ref[...] * (y_ref[...] - mu) * inv + b_ref[...]
        xn = jnp.tanh(yb)
        seg = seg_ref[...]
        wsel = jnp.zeros((BR, H), jnp.float32)
        bsel = jnp.zeros((BR, 1), jnp.float32)
        for c in range(C):
            m = jnp.where(seg == c, 1.0, 0.0)
            wsel = wsel + m * wf_ref[c:c + 1, :]
            bsel = bsel + m * bf_ref[c, 0]
        v = jnp.sum(xn * wsel, axis=1, keepdims=True) + bsel
        rows = i * BR + lax.broadcasted_iota(jnp.int32, (BR, 1), 0)
        v = jnp.where(rows < n_real, v, 0.0)
        v_ref[...] = v
        sv1 = jnp.sum(v)
        sv2 = jnp.sum(v * v)
        r8 = lax.broadcasted_iota(jnp.int32, (8, 128), 0)
        c8 = lax.broadcasted_iota(jnp.int32, (8, 128), 1)
        sblk = (jnp.where((r8 == 0) & (c8 == 0), sv1, 0.0)
                + jnp.where((r8 == 1) & (c8 == 0), sv2, 0.0))

        @pl.when(i == 0)
        def _():
            vs_ref[...] = sblk

        @pl.when(i > 0)
        def _():
            vs_ref[...] = vs_ref[...] + sblk

    return pl.pallas_call(
        body,
        grid=(G,),
        in_specs=[
            pl.BlockSpec((BR, H), lambda i: (i, 0)),
            pl.BlockSpec((8, H), lambda i: (0, 0)),
            pl.BlockSpec((1, H), lambda i: (0, 0)),
            pl.BlockSpec((1, H), lambda i: (0, 0)),
            pl.BlockSpec((C, H), lambda i: (0, 0)),
            pl.BlockSpec((C, 1), lambda i: (0, 0)),
            pl.BlockSpec((BR, 1), lambda i: (i, 0)),
        ],
        out_specs=[
            pl.BlockSpec((BR, 1), lambda i: (i, 0)),
            pl.BlockSpec((8, 128), lambda i: (0, 0)),
        ],
        out_shape=[
            jax.ShapeDtypeStruct((NP, 1), jnp.float32),
            jax.ShapeDtypeStruct((8, 128), jnp.float32),
        ],
    )(y, sums, g, b, wfc, bfc, seg)


def _tc_final(v, vsums, lng, lnb, n_real):
    NP = v.shape[0]

    def body(v_ref, vs_ref, g_ref, b_ref, o_ref):
        s1 = vs_ref[0:1, 0:1]
        s2 = vs_ref[1:2, 0:1]
        mu = s1 / n_real
        var = s2 / n_real - mu * mu
        inv = lax.rsqrt(var + 1e-5)
        o_ref[...] = g_ref[...] * (v_ref[...] - mu) * inv + b_ref[...]

    return pl.pallas_call(
        body,
        in_specs=[
            pl.BlockSpec((NP, 1), lambda: (0, 0)),
            pl.BlockSpec((8, 128), lambda: (0, 0)),
            pl.BlockSpec((NP, 1), lambda: (0, 0)),
            pl.BlockSpec((NP, 1), lambda: (0, 0)),
        ],
        out_specs=pl.BlockSpec((NP, 1), lambda: (0, 0)),
        out_shape=jax.ShapeDtypeStruct((NP, 1), jnp.float32),
    )(v, vsums, lng, lnb)


def kernel(features, adj, segment, W1, g1, b1, W2, g2, b2, Wf, bf, lng, lnb):
    N, F = features.shape
    E = adj.shape[1]
    C, K, _, H = W1.shape

    NP = (N // 256 + 1) * 256
    PAD_ROWS = NP - N
    EP = ((E + LANE * NW - 1) // (LANE * NW)) * (LANE * NW)
    CPW = EP // (LANE * NW)
    RPT = NP // NS
    BR = 1024 if NP % 1024 == 0 else 512

    f32 = jnp.float32

    pad_idx = N + (jnp.arange(EP - E, dtype=jnp.int32) % PAD_ROWS)
    src2d = jnp.concatenate([adj[0], pad_idx]).reshape(EP // LANE, LANE)
    dst2d = jnp.concatenate([adj[1], pad_idx]).reshape(EP // LANE, LANE)
    zpad_rows = jnp.zeros((PAD_ROWS, F), f32)
    x_pad = jnp.concatenate([features.astype(f32), zpad_rows])
    seg_pad = jnp.concatenate([segment, jnp.zeros((PAD_ROWS,), jnp.int32)])
    seg_pad = seg_pad.reshape(NP, 1)
    zrows = jnp.zeros((RPT, F), f32)
    zrow1 = jnp.zeros((RPT,), f32)
    ones_row = jnp.ones((LANE,), f32)
    W1c = W1.reshape(C, K * F, H).astype(f32)
    W2c = W2.reshape(C, K * H, H).astype(f32)
    g1r = g1.reshape(1, H)
    b1r = b1.reshape(1, H)
    g2r = g2.reshape(1, H)
    b2r = b2.reshape(1, H)
    wfc = Wf[:, :, 0].astype(f32)
    lng_p = jnp.concatenate([lng, jnp.zeros((PAD_ROWS,), f32)]).reshape(NP, 1)
    lnb_p = jnp.concatenate([lnb, jnp.zeros((PAD_ROWS,), f32)]).reshape(NP, 1)
    n_real = float(N)

    degp = _sc_deg(dst2d, ones_row, zrow1, NP, CPW)
    deg0 = degp[0].reshape(NP, 1)
    deg1 = degp[1].reshape(NP, 1)

    u0, dinv = _tc_pre(deg0, deg1, x_pad, BR)

    def cheb_layer(x_in, u_in, wcat):
        p = _sc_prop(u_in, src2d, dst2d, zrows, NP, CPW)
        tx1, u1 = _tc_mid(p[0], p[1], dinv, BR)
        q = _sc_prop(u1, src2d, dst2d, zrows, NP, CPW)
        return _tc_layer(q[0], q[1], dinv, x_in, tx1, wcat, seg_pad, BR)

    y1, s1 = cheb_layer(x_pad, u0, W1c)
    x1, u1b = _tc_post1(y1, s1, g1r, b1r, dinv, n_real, BR)
    y2, s2 = cheb_layer(x1, u1b, W2c)
    v, vs = _tc_post2(y2, s2, g2r, b2r, wfc, bf.astype(f32), seg_pad, n_real, BR)
    o = _tc_final(v, vs, lng_p, lnb_p, n_real)
    return o.reshape(-1)[:N]


# speedup vs baseline: 14.8582x; 1.7079x over previous
import functools

import jax
import jax.numpy as jnp
from jax import lax
from jax.experimental import pallas as pl
from jax.experimental.pallas import tpu as pltpu
from jax.experimental.pallas import tpu_sc as plsc

NC = 2
NS = 16
NW = NC * NS
LANE = 128


def _sc_mesh():
    return plsc.VectorSubcoreMesh(
        core_axis_name="c", subcore_axis_name="s", num_cores=NC, num_subcores=NS
    )


NBUF = 2


def _sc_deg(idx2, ones_row, zrow, NP, CPW):
    RPT = NP // NS

    @functools.partial(
        pl.kernel,
        out_type=jax.ShapeDtypeStruct((NC, NP), jnp.float32),
        mesh=_sc_mesh(),
        scratch_types=[
            pltpu.VMEM((NBUF, LANE), jnp.int32),
            pltpu.VMEM((LANE,), jnp.float32),
            pltpu.VMEM_SHARED((NP,), jnp.float32),
            pltpu.SemaphoreType.DMA((NBUF,)),
        ],
    )
    def k(idx_hbm, ones_hbm, z_hbm, out_hbm, dst_v, ones_v, acc, isem):
        ci = lax.axis_index("c")
        si = lax.axis_index("s")
        wid = si * NC + ci
        pltpu.sync_copy(z_hbm, acc.at[pl.ds(si * RPT, RPT)])
        pltpu.sync_copy(ones_hbm, ones_v)
        plsc.subcore_barrier()
        base = wid * CPW
        for b in range(NBUF):
            pltpu.async_copy(idx_hbm.at[base + b, 1], dst_v.at[b], isem.at[b])

        def group(g, carry):
            j = base + g * NBUF
            for b in range(NBUF):
                pltpu.make_async_copy(
                    idx_hbm.at[j + b, 1], dst_v.at[b], isem.at[b]).wait()
                pltpu.sync_copy(ones_v, acc.at[dst_v.at[b]], add=True)
                pltpu.async_copy(
                    idx_hbm.at[j + b + NBUF, 1], dst_v.at[b], isem.at[b])
            return carry

        lax.fori_loop(0, CPW // NBUF - 1, group, 0)
        j = base + CPW - NBUF
        for b in range(NBUF):
            pltpu.make_async_copy(
                idx_hbm.at[j + b, 1], dst_v.at[b], isem.at[b]).wait()
            pltpu.sync_copy(ones_v, acc.at[dst_v.at[b]], add=True)
        plsc.subcore_barrier()
        pltpu.sync_copy(acc.at[pl.ds(si * RPT, RPT)],
                        out_hbm.at[ci, pl.ds(si * RPT, RPT)])

    return k(idx2, ones_row, zrow)


def _sc_prop(y, idx2, zrows, NP, CPW):
    RPT = NP // NS
    F = y.shape[1]

    @functools.partial(
        pl.kernel,
        out_type=jax.ShapeDtypeStruct((NC, NP, F), jnp.float32),
        mesh=_sc_mesh(),
        scratch_types=[
            pltpu.VMEM((NBUF, 2, LANE), jnp.int32),
            pltpu.VMEM((NBUF, LANE, F), jnp.float32),
            pltpu.VMEM_SHARED((NP, F), jnp.float32),
            pltpu.SemaphoreType.DMA((NBUF,)),
        ],
    )
    def k(y_hbm, idx_hbm, z_hbm, out_hbm, idx_v, rows_v, acc, gsem):
        ci = lax.axis_index("c")
        si = lax.axis_index("s")
        wid = si * NC + ci
        pltpu.sync_copy(z_hbm, acc.at[pl.ds(si * RPT, RPT)])
        plsc.subcore_barrier()
        base = wid * CPW
        for b in range(NBUF):
            pltpu.sync_copy(idx_hbm.at[base + b], idx_v.at[b])
            pltpu.async_copy(y_hbm.at[idx_v.at[b, 0]], rows_v.at[b], gsem.at[b])

        def group(g, carry):
            j = base + g * NBUF
            for b in range(NBUF):
                pltpu.make_async_copy(
                    y_hbm.at[idx_v.at[b, 0]], rows_v.at[b], gsem.at[b]).wait()
                pltpu.sync_copy(rows_v.at[b], acc.at[idx_v.at[b, 1]], add=True)
                pltpu.sync_copy(idx_hbm.at[j + b + NBUF], idx_v.at[b])
                pltpu.async_copy(y_hbm.at[idx_v.at[b, 0]], rows_v.at[b],
                                 gsem.at[b])
            return carry

        lax.fori_loop(0, CPW // NBUF - 1, group, 0)
        for b in range(NBUF):
            pltpu.make_async_copy(
                y_hbm.at[idx_v.at[b, 0]], rows_v.at[b], gsem.at[b]).wait()
            pltpu.sync_copy(rows_v.at[b], acc.at[idx_v.at[b, 1]], add=True)
        plsc.subcore_barrier()
        pltpu.sync_copy(acc.at[pl.ds(si * RPT, RPT)],
                        out_hbm.at[ci, pl.ds(si * RPT, RPT)])

    return k(y, idx2, zrows)


def _tc_pre(deg0, deg1, x, BR):
    NP, F = x.shape
    G = NP // BR

    def body(d0_ref, d1_ref, x_ref, u_ref, dinv_ref):
        deg = jnp.maximum(d0_ref[...] + d1_ref[...], 1.0)
        dinv = lax.rsqrt(deg)
        dinv_ref[...] = dinv
        u_ref[...] = x_ref[...] * dinv

    return pl.pallas_call(
        body,
        grid=(G,),
        in_specs=[
            pl.BlockSpec((BR, 1), lambda i: (i, 0)),
            pl.BlockSpec((BR, 1), lambda i: (i, 0)),
            pl.BlockSpec((BR, F), lambda i: (i, 0)),
        ],
        out_specs=[
            pl.BlockSpec((BR, F), lambda i: (i, 0)),
            pl.BlockSpec((BR, 1), lambda i: (i, 0)),
        ],
        out_shape=[
            jax.ShapeDtypeStruct((NP, F), jnp.float32),
            jax.ShapeDtypeStruct((NP, 1), jnp.float32),
        ],
    )(deg0, deg1, x)


def _tc_mid(pp0, pp1, dinv, BR):
    NP, F = pp0.shape
    G = NP // BR

    def body(p0_ref, p1_ref, dv_ref, tx1_ref, u_ref):
        dv = dv_ref[...]
        tx1 = -dv * (p0_ref[...] + p1_ref[...])
        tx1_ref[...] = tx1
        u_ref[...] = dv * tx1

    return pl.pallas_call(
        body,
        grid=(G,),
        in_specs=[
            pl.BlockSpec((BR, F), lambda i: (i, 0)),
            pl.BlockSpec((BR, F), lambda i: (i, 0)),
            pl.BlockSpec((BR, 1), lambda i: (i, 0)),
        ],
        out_specs=[
            pl.BlockSpec((BR, F), lambda i: (i, 0)),
            pl.BlockSpec((BR, F), lambda i: (i, 0)),
        ],
        out_shape=[
            jax.ShapeDtypeStruct((NP, F), jnp.float32),
            jax.ShapeDtypeStruct((NP, F), jnp.float32),
        ],
    )(pp0, pp1, dinv)


def _tc_layer(pp0, pp1, dinv, x_in, tx1, wcat, seg, BR):
    NP, F = x_in.shape
    C, D3, H = wcat.shape
    G = NP // BR

    def body(p0_ref, p1_ref, dv_ref, x_ref, t1_ref, w_ref, seg_ref, y_ref, s_ref):
        i = pl.program_id(0)
        dv = dv_ref[...]
        x = x_ref[...]
        t1 = t1_ref[...]
        t2 = -2.0 * dv * (p0_ref[...] + p1_ref[...]) - x
        t = jnp.concatenate([x, t1, t2], axis=1)
        seg = seg_ref[...]
        y = jnp.zeros((BR, H), jnp.float32)
        for c in range(C):
            m = jnp.where(seg == c, 1.0, 0.0)
            y = y + m * jnp.dot(t, w_ref[c], preferred_element_type=jnp.float32)
        y_ref[...] = y
        s1 = jnp.sum(y, axis=0, keepdims=True)
        s2 = jnp.sum(y * y, axis=0, keepdims=True)
        rows = lax.broadcasted_iota(jnp.int32, (8, H), 0)
        sblk = (jnp.where(rows == 0, jnp.broadcast_to(s1, (8, H)), 0.0)
                + jnp.where(rows == 1, jnp.broadcast_to(s2, (8, H)), 0.0))

        @pl.when(i == 0)
        def _():
            s_ref[...] = sblk

        @pl.when(i > 0)
        def _():
            s_ref[...] = s_ref[...] + sblk

    return pl.pallas_call(
        body,
        grid=(G,),
        in_specs=[
            pl.BlockSpec((BR, F), lambda i: (i, 0)),
            pl.BlockSpec((BR, F), lambda i: (i, 0)),
            pl.BlockSpec((BR, 1), lambda i: (i, 0)),
            pl.BlockSpec((BR, F), lambda i: (i, 0)),
            pl.BlockSpec((BR, F), lambda i: (i, 0)),
            pl.BlockSpec((C, D3, H), lambda i: (0, 0, 0)),
            pl.BlockSpec((BR, 1), lambda i: (i, 0)),
        ],
        out_specs=[
            pl.BlockSpec((BR, H), lambda i: (i, 0)),
            pl.BlockSpec((8, H), lambda i: (0, 0)),
        ],
        out_shape=[
            jax.ShapeDtypeStruct((NP, H), jnp.float32),
            jax.ShapeDtypeStruct((8, H), jnp.float32),
        ],
    )(pp0, pp1, dinv, x_in, tx1, wcat, seg)


def _tc_post1(y, sums, g, b, dinv, n_real, BR):
    NP, H = y.shape
    G = NP // BR

    def body(y_ref, s_ref, g_ref, b_ref, dv_ref, x_ref, u_ref):
        i = pl.program_id(0)
        s1 = s_ref[0:1, :]
        s2 = s_ref[1:2, :]
        mu = s1 / n_real
        var = s2 / n_real - mu * mu
        inv = lax.rsqrt(var + 1e-5)
        yb = g_ref[...] * (y_ref[...] - mu) * inv + b_ref[...]
        xn = yb * (1.0 / (1.0 + jnp.exp(-yb)))
        rows = i * BR + lax.broadcasted_iota(jnp.int32, (BR, 1), 0)
        xn = jnp.where(rows < n_real, xn, 0.0)
        x_ref[...] = xn
        u_ref[...] = dv_ref[...] * xn

    return pl.pallas_call(
        body,
        grid=(G,),
        in_specs=[
            pl.BlockSpec((BR, H), lambda i: (i, 0)),
            pl.BlockSpec((8, H), lambda i: (0, 0)),
            pl.BlockSpec((1, H), lambda i: (0, 0)),
            pl.BlockSpec((1, H), lambda i: (0, 0)),
            pl.BlockSpec((BR, 1), lambda i: (i, 0)),
        ],
        out_specs=[
            pl.BlockSpec((BR, H), lambda i: (i, 0)),
            pl.BlockSpec((BR, H), lambda i: (i, 0)),
        ],
        out_shape=[
            jax.ShapeDtypeStruct((NP, H), jnp.float32),
            jax.ShapeDtypeStruct((NP, H), jnp.float32),
        ],
    )(y, sums, g, b, dinv)


def _tc_post2(y, sums, g, b, wfc, bfc, seg, n_real, BR):
    NP, H = y.shape
    C = wfc.shape[0]
    G = NP // BR

    def body(y_ref, s_ref, g_ref, b_ref, wf_ref, bf_ref, seg_ref, v_ref, vs_ref):
        i = pl.program_id(0)
        s1 = s_ref[0:1, :]
        s2 = s_ref[1:2, :]
        mu = s1 / n_real
        var = s2 / n_real - mu * mu
        inv = lax.rsqrt(var + 1e-5)
        yb = g_ref[...] * (y_ref[...] - mu) * inv + b_ref[...]
        xn = jnp.tanh(yb)
        seg = seg_ref[...]
        wsel = jnp.zeros((BR, H), jnp.float32)
        bsel = jnp.zeros((BR, 1), jnp.float32)
        for c in range(C):
            m = jnp.where(seg == c, 1.0, 0.0)
            wsel = wsel + m * wf_ref[c:c + 1, :]
            bsel = bsel + m * bf_ref[c, 0]
        v = jnp.sum(xn * wsel, axis=1, keepdims=True) + bsel
        rows = i * BR + lax.broadcasted_iota(jnp.int32, (BR, 1), 0)
        v = jnp.where(rows < n_real, v, 0.0)
        v_ref[...] = v
        sv1 = jnp.sum(v)
        sv2 = jnp.sum(v * v)
        r8 = lax.broadcasted_iota(jnp.int32, (8, 128), 0)
        c8 = lax.broadcasted_iota(jnp.int32, (8, 128), 1)
        sblk = (jnp.where((r8 == 0) & (c8 == 0), sv1, 0.0)
                + jnp.where((r8 == 1) & (c8 == 0), sv2, 0.0))

        @pl.when(i == 0)
        def _():
            vs_ref[...] = sblk

        @pl.when(i > 0)
        def _():
            vs_ref[...] = vs_ref[...] + sblk

    return pl.pallas_call(
        body,
        grid=(G,),
        in_specs=[
            pl.BlockSpec((BR, H), lambda i: (i, 0)),
            pl.BlockSpec((8, H), lambda i: (0, 0)),
            pl.BlockSpec((1, H), lambda i: (0, 0)),
            pl.BlockSpec((1, H), lambda i: (0, 0)),
            pl.BlockSpec((C, H), lambda i: (0, 0)),
            pl.BlockSpec((C, 1), lambda i: (0, 0)),
            pl.BlockSpec((BR, 1), lambda i: (i, 0)),
        ],
        out_specs=[
            pl.BlockSpec((BR, 1), lambda i: (i, 0)),
            pl.BlockSpec((8, 128), lambda i: (0, 0)),
        ],
        out_shape=[
            jax.ShapeDtypeStruct((NP, 1), jnp.float32),
            jax.ShapeDtypeStruct((8, 128), jnp.float32),
        ],
    )(y, sums, g, b, wfc, bfc, seg)


def _tc_final(v, vsums, lng, lnb, n_real):
    NP = v.shape[0]

    def body(v_ref, vs_ref, g_ref, b_ref, o_ref):
        s1 = vs_ref[0:1, 0:1]
        s2 = vs_ref[1:2, 0:1]
        mu = s1 / n_real
        var = s2 / n_real - mu * mu
        inv = lax.rsqrt(var + 1e-5)
        o_ref[...] = g_ref[...] * (v_ref[...] - mu) * inv + b_ref[...]

    return pl.pallas_call(
        body,
        in_specs=[
            pl.BlockSpec((NP, 1), lambda: (0, 0)),
            pl.BlockSpec((8, 128), lambda: (0, 0)),
            pl.BlockSpec((NP, 1), lambda: (0, 0)),
            pl.BlockSpec((NP, 1), lambda: (0, 0)),
        ],
        out_specs=pl.BlockSpec((NP, 1), lambda: (0, 0)),
        out_shape=jax.ShapeDtypeStruct((NP, 1), jnp.float32),
    )(v, vsums, lng, lnb)


def kernel(features, adj, segment, W1, g1, b1, W2, g2, b2, Wf, bf, lng, lnb):
    N, F = features.shape
    E = adj.shape[1]
    C, K, _, H = W1.shape

    NP = (N // 256 + 1) * 256
    PAD_ROWS = NP - N
    EQ = LANE * NW * NBUF
    EP = ((E + EQ - 1) // EQ) * EQ
    CPW = EP // (LANE * NW)
    RPT = NP // NS
    BR = 1024 if NP % 1024 == 0 else 512

    f32 = jnp.float32

    pad_idx = N + (jnp.arange(EP - E, dtype=jnp.int32) % PAD_ROWS)
    src2d = jnp.concatenate([adj[0], pad_idx]).reshape(EP // LANE, LANE)
    dst2d = jnp.concatenate([adj[1], pad_idx]).reshape(EP // LANE, LANE)
    idx2 = jnp.stack([src2d, dst2d], axis=1)
    zpad_rows = jnp.zeros((PAD_ROWS, F), f32)
    x_pad = jnp.concatenate([features.astype(f32), zpad_rows])
    seg_pad = jnp.concatenate([segment, jnp.zeros((PAD_ROWS,), jnp.int32)])
    seg_pad = seg_pad.reshape(NP, 1)
    zrows = jnp.zeros((RPT, F), f32)
    zrow1 = jnp.zeros((RPT,), f32)
    ones_row = jnp.ones((LANE,), f32)
    W1c = W1.reshape(C, K * F, H).astype(f32)
    W2c = W2.reshape(C, K * H, H).astype(f32)
    g1r = g1.reshape(1, H)
    b1r = b1.reshape(1, H)
    g2r = g2.reshape(1, H)
    b2r = b2.reshape(1, H)
    wfc = Wf[:, :, 0].astype(f32)
    lng_p = jnp.concatenate([lng, jnp.zeros((PAD_ROWS,), f32)]).reshape(NP, 1)
    lnb_p = jnp.concatenate([lnb, jnp.zeros((PAD_ROWS,), f32)]).reshape(NP, 1)
    n_real = float(N)

    degp = _sc_deg(idx2, ones_row, zrow1, NP, CPW)
    deg0 = degp[0].reshape(NP, 1)
    deg1 = degp[1].reshape(NP, 1)

    u0, dinv = _tc_pre(deg0, deg1, x_pad, BR)

    def cheb_layer(x_in, u_in, wcat):
        p = _sc_prop(u_in, idx2, zrows, NP, CPW)
        tx1, u1 = _tc_mid(p[0], p[1], dinv, BR)
        q = _sc_prop(u1, idx2, zrows, NP, CPW)
        return _tc_layer(q[0], q[1], dinv, x_in, tx1, wcat, seg_pad, BR)

    y1, s1 = cheb_layer(x_pad, u0, W1c)
    x1, u1b = _tc_post1(y1, s1, g1r, b1r, dinv, n_real, BR)
    y2, s2 = cheb_layer(x1, u1b, W2c)
    v, vs = _tc_post2(y2, s2, g2r, b2r, wfc, bf.astype(f32), seg_pad, n_real, BR)
    o = _tc_final(v, vs, lng_p, lnb_p, n_real)
    return o.reshape(-1)[:N]


# speedup vs baseline: 15.1331x; 1.0185x over previous
import functools

import jax
import jax.numpy as jnp
from jax import lax
from jax.experimental import pallas as pl
from jax.experimental.pallas import tpu as pltpu
from jax.experimental.pallas import tpu_sc as plsc

NC = 2
NS = 16
NW = NC * NS
LANE = 128
NBUF = 4


def _sc_mesh():
    return plsc.VectorSubcoreMesh(
        core_axis_name="c", subcore_axis_name="s", num_cores=NC, num_subcores=NS
    )


def _sc_deg(idx3, ones_row, zrow, NP, CPW):
    RPT = NP // NS

    @functools.partial(
        pl.kernel,
        out_type=jax.ShapeDtypeStruct((NC, NP), jnp.float32),
        mesh=_sc_mesh(),
        scratch_types=[
            pltpu.VMEM((NBUF, LANE), jnp.int32),
            pltpu.VMEM((LANE,), jnp.float32),
            pltpu.VMEM_SHARED((NP,), jnp.float32),
            pltpu.SemaphoreType.DMA((NBUF,)),
        ],
    )
    def k(idx_hbm, ones_hbm, z_hbm, out_hbm, dst_v, ones_v, acc, isem):
        ci = lax.axis_index("c")
        si = lax.axis_index("s")
        wid = si * NC + ci
        pltpu.sync_copy(z_hbm, acc.at[pl.ds(si * RPT, RPT)])
        pltpu.sync_copy(ones_hbm, ones_v)
        plsc.subcore_barrier()
        base = wid * CPW
        for b in range(NBUF):
            pltpu.async_copy(idx_hbm.at[(base + b) * 2 + 1], dst_v.at[b],
                             isem.at[b])

        def group(g, carry):
            j = base + g * NBUF
            for b in range(NBUF):
                pltpu.make_async_copy(
                    idx_hbm.at[(j + b) * 2 + 1], dst_v.at[b], isem.at[b]).wait()
                pltpu.sync_copy(ones_v, acc.at[dst_v.at[b]], add=True)
                pltpu.async_copy(
                    idx_hbm.at[(j + b + NBUF) * 2 + 1], dst_v.at[b], isem.at[b])
            return carry

        lax.fori_loop(0, CPW // NBUF - 1, group, 0)
        j = base + CPW - NBUF
        for b in range(NBUF):
            pltpu.make_async_copy(
                idx_hbm.at[(j + b) * 2 + 1], dst_v.at[b], isem.at[b]).wait()
            pltpu.sync_copy(ones_v, acc.at[dst_v.at[b]], add=True)
        plsc.subcore_barrier()
        pltpu.sync_copy(acc.at[pl.ds(si * RPT, RPT)],
                        out_hbm.at[ci, pl.ds(si * RPT, RPT)])

    return k(idx3, ones_row, zrow)


def _sc_prop(y, idx2, zrows, NP, CPW):
    RPT = NP // NS
    F = y.shape[1]
    NI = 4

    @functools.partial(
        pl.kernel,
        out_type=jax.ShapeDtypeStruct((NC, NP, F), jnp.float32),
        mesh=_sc_mesh(),
        scratch_types=[
            pltpu.VMEM((NI, 2, LANE), jnp.int32),
            pltpu.VMEM((2, LANE, F), jnp.float32),
            pltpu.VMEM_SHARED((NP, F), jnp.float32),
            pltpu.SemaphoreType.DMA((2,)),
            pltpu.SemaphoreType.DMA((2,)),
            pltpu.SemaphoreType.DMA((NI,)),
        ],
    )
    def k(y_hbm, idx_hbm, z_hbm, out_hbm, idx_v, rows_v, acc, gsem, ssem, isem):
        ci = lax.axis_index("c")
        si = lax.axis_index("s")
        wid = si * NC + ci
        pltpu.sync_copy(z_hbm, acc.at[pl.ds(si * RPT, RPT)])
        plsc.subcore_barrier()
        base = wid * CPW

        def fetch_idx(ch, q):
            pltpu.async_copy(idx_hbm.at[ch * 2], idx_v.at[q, 0], isem.at[q])
            pltpu.async_copy(idx_hbm.at[ch * 2 + 1], idx_v.at[q, 1], isem.at[q])

        def wait_idx(q):
            for r in range(2):
                pltpu.make_async_copy(
                    idx_hbm.at[0], idx_v.at[q, r], isem.at[q]).wait()

        def issue_gather(q, b):
            pltpu.async_copy(y_hbm.at[idx_v.at[q, 0]], rows_v.at[b],
                             gsem.at[b])

        def wait_gather(q, b):
            pltpu.make_async_copy(y_hbm.at[idx_v.at[q, 0]], rows_v.at[b],
                                  gsem.at[b]).wait()

        def issue_scatter(q, b):
            pltpu.async_copy(rows_v.at[b], acc.at[idx_v.at[q, 1]], ssem.at[b],
                             add=True)

        def wait_scatter(q, b):
            pltpu.make_async_copy(rows_v.at[b], acc.at[idx_v.at[q, 1]],
                                  ssem.at[b]).wait()

        def step(j, jm2, jm4, do_c, do_d, do_ef):
            b, b2 = jm2, 1 - jm2
            q1, q3 = (jm4 + 1) % NI, (jm4 + 3) % NI
            wait_gather(jm4, b)
            issue_scatter(jm4, b)
            if do_c:
                wait_scatter(q3, b2)
            if do_d:
                fetch_idx(j + 3, q3)
            if do_ef:
                wait_idx(q1)
                issue_gather(q1, b2)

        fetch_idx(base, 0)
        fetch_idx(base + 1, 1)
        fetch_idx(base + 2, 2)
        wait_idx(0)
        issue_gather(0, 0)
        step(base + 0, 0, 0, False, True, True)
        step(base + 1, 1, 1, True, True, True)
        step(base + 2, 0, 2, True, True, True)
        step(base + 3, 1, 3, True, True, True)

        def group(g, carry):
            j = base + 4 + g * 4
            for t in range(4):
                step(j + t, t % 2, t, True, True, True)
            return carry

        lax.fori_loop(0, (CPW - 8) // 4, group, 0)
        jt = base + CPW - 4
        step(jt + 0, 0, 0, True, True, True)
        step(jt + 1, 1, 1, True, False, True)
        step(jt + 2, 0, 2, True, False, True)
        step(jt + 3, 1, 3, True, False, False)
        wait_scatter(3, 1)
        plsc.subcore_barrier()
        pltpu.sync_copy(acc.at[pl.ds(si * RPT, RPT)],
                        out_hbm.at[ci, pl.ds(si * RPT, RPT)])

    return k(y, idx2, zrows)


def _tc_pre(deg0, deg1, x, BR):
    NP, F = x.shape
    G = NP // BR

    def body(d0_ref, d1_ref, x_ref, u_ref, dinv_ref):
        deg = jnp.maximum(d0_ref[...] + d1_ref[...], 1.0)
        dinv = lax.rsqrt(deg)
        dinv_ref[...] = dinv
        u_ref[...] = x_ref[...] * dinv

    return pl.pallas_call(
        body,
        grid=(G,),
        in_specs=[
            pl.BlockSpec((BR, 1), lambda i: (i, 0)),
            pl.BlockSpec((BR, 1), lambda i: (i, 0)),
            pl.BlockSpec((BR, F), lambda i: (i, 0)),
        ],
        out_specs=[
            pl.BlockSpec((BR, F), lambda i: (i, 0)),
            pl.BlockSpec((BR, 1), lambda i: (i, 0)),
        ],
        out_shape=[
            jax.ShapeDtypeStruct((NP, F), jnp.float32),
            jax.ShapeDtypeStruct((NP, 1), jnp.float32),
        ],
    )(deg0, deg1, x)


def _tc_mid(p2, dinv, BR):
    NP2, F = p2.shape
    NP = NP2 // 2
    G = NP // BR

    def body(pa_ref, pb_ref, dv_ref, tx1_ref, u_ref):
        dv = dv_ref[...]
        tx1 = -dv * (pa_ref[...] + pb_ref[...])
        tx1_ref[...] = tx1
        u_ref[...] = dv * tx1

    return pl.pallas_call(
        body,
        grid=(G,),
        in_specs=[
            pl.BlockSpec((BR, F), lambda i: (i, 0)),
            pl.BlockSpec((BR, F), lambda i: (G + i, 0)),
            pl.BlockSpec((BR, 1), lambda i: (i, 0)),
        ],
        out_specs=[
            pl.BlockSpec((BR, F), lambda i: (i, 0)),
            pl.BlockSpec((BR, F), lambda i: (i, 0)),
        ],
        out_shape=[
            jax.ShapeDtypeStruct((NP, F), jnp.float32),
            jax.ShapeDtypeStruct((NP, F), jnp.float32),
        ],
    )(p2, p2, dinv)


def _tc_layer(q2, dinv, x_in, tx1, wcat, seg, BR):
    NP, F = x_in.shape
    C, D3, H = wcat.shape
    G = NP // BR

    def body(qa_ref, qb_ref, dv_ref, x_ref, t1_ref, w_ref, seg_ref, y_ref, s_ref):
        i = pl.program_id(0)
        dv = dv_ref[...]
        x = x_ref[...]
        t1 = t1_ref[...]
        q = qa_ref[...] + qb_ref[...]
        t2 = -2.0 * dv * q - x
        t = jnp.concatenate([x, t1, t2], axis=1)
        seg = seg_ref[...]
        y = jnp.zeros((BR, H), jnp.float32)
        for c in range(C):
            m = jnp.where(seg == c, 1.0, 0.0)
            y = y + m * jnp.dot(t, w_ref[c], preferred_element_type=jnp.float32)
        y_ref[...] = y
        s1 = jnp.sum(y, axis=0, keepdims=True)
        s2 = jnp.sum(y * y, axis=0, keepdims=True)
        rows = lax.broadcasted_iota(jnp.int32, (8, H), 0)
        sblk = (jnp.where(rows == 0, jnp.broadcast_to(s1, (8, H)), 0.0)
                + jnp.where(rows == 1, jnp.broadcast_to(s2, (8, H)), 0.0))

        @pl.when(i == 0)
        def _():
            s_ref[...] = sblk

        @pl.when(i > 0)
        def _():
            s_ref[...] = s_ref[...] + sblk

    return pl.pallas_call(
        body,
        grid=(G,),
        in_specs=[
            pl.BlockSpec((BR, F), lambda i: (i, 0)),
            pl.BlockSpec((BR, F), lambda i: (G + i, 0)),
            pl.BlockSpec((BR, 1), lambda i: (i, 0)),
            pl.BlockSpec((BR, F), lambda i: (i, 0)),
            pl.BlockSpec((BR, F), lambda i: (i, 0)),
            pl.BlockSpec((C, D3, H), lambda i: (0, 0, 0)),
            pl.BlockSpec((BR, 1), lambda i: (i, 0)),
        ],
        out_specs=[
            pl.BlockSpec((BR, H), lambda i: (i, 0)),
            pl.BlockSpec((8, H), lambda i: (0, 0)),
        ],
        out_shape=[
            jax.ShapeDtypeStruct((NP, H), jnp.float32),
            jax.ShapeDtypeStruct((8, H), jnp.float32),
        ],
    )(q2, q2, dinv, x_in, tx1, wcat, seg)


def _tc_post1(y, sums, g, b, dinv, n_real, BR):
    NP, H = y.shape
    G = NP // BR

    def body(y_ref, s_ref, g_ref, b_ref, dv_ref, x_ref, u_ref):
        i = pl.program_id(0)
        s1 = s_ref[0:1, :]
        s2 = s_ref[1:2, :]
        mu = s1 / n_real
        var = s2 / n_real - mu * mu
        inv = lax.rsqrt(var + 1e-5)
        yb = g_ref[...] * (y_ref[...] - mu) * inv + b_ref[...]
        xn = yb * (1.0 / (1.0 + jnp.exp(-yb)))
        rows = i * BR + lax.broadcasted_iota(jnp.int32, (BR, 1), 0)
        xn = jnp.where(rows < n_real, xn, 0.0)
        x_ref[...] = xn
        u_ref[...] = dv_ref[...] * xn

    return pl.pallas_call(
        body,
        grid=(G,),
        in_specs=[
            pl.BlockSpec((BR, H), lambda i: (i, 0)),
            pl.BlockSpec((8, H), lambda i: (0, 0)),
            pl.BlockSpec((1, H), lambda i: (0, 0)),
            pl.BlockSpec((1, H), lambda i: (0, 0)),
            pl.BlockSpec((BR, 1), lambda i: (i, 0)),
        ],
        out_specs=[
            pl.BlockSpec((BR, H), lambda i: (i, 0)),
            pl.BlockSpec((BR, H), lambda i: (i, 0)),
        ],
        out_shape=[
            jax.ShapeDtypeStruct((NP, H), jnp.float32),
            jax.ShapeDtypeStruct((NP, H), jnp.float32),
        ],
    )(y, sums, g, b, dinv)


def _tc_post2(y, sums, g, b, wfc, bfc, seg, n_real, BR):
    NP, H = y.shape
    C = wfc.shape[0]
    G = NP // BR

    def body(y_ref, s_ref, g_ref, b_ref, wf_ref, bf_ref, seg_ref, v_ref, vs_ref):
        i = pl.program_id(0)
        s1 = s_ref[0:1, :]
        s2 = s_ref[1:2, :]
        mu = s1 / n_real
        var = s2 / n_real - mu * mu
        inv = lax.rsqrt(var + 1e-5)
        yb = g_ref[...] * (y_ref[...] - mu) * inv + b_ref[...]
        xn = jnp.tanh(yb)
        seg = seg_ref[...]
        wsel = jnp.zeros((BR, H), jnp.float32)
        bsel = jnp.zeros((BR, 1), jnp.float32)
        for c in range(C):
            m = jnp.where(seg == c, 1.0, 0.0)
            wsel = wsel + m * wf_ref[c:c + 1, :]
            bsel = bsel + m * bf_ref[c, 0]
        v = jnp.sum(xn * wsel, axis=1, keepdims=True) + bsel
        rows = i * BR + lax.broadcasted_iota(jnp.int32, (BR, 1), 0)
        v = jnp.where(rows < n_real, v, 0.0)
        v_ref[...] = v
        sv1 = jnp.sum(v)
        sv2 = jnp.sum(v * v)
        r8 = lax.broadcasted_iota(jnp.int32, (8, 128), 0)
        c8 = lax.broadcasted_iota(jnp.int32, (8, 128), 1)
        sblk = (jnp.where((r8 == 0) & (c8 == 0), sv1, 0.0)
                + jnp.where((r8 == 1) & (c8 == 0), sv2, 0.0))

        @pl.when(i == 0)
        def _():
            vs_ref[...] = sblk

        @pl.when(i > 0)
        def _():
            vs_ref[...] = vs_ref[...] + sblk

    return pl.pallas_call(
        body,
        grid=(G,),
        in_specs=[
            pl.BlockSpec((BR, H), lambda i: (i, 0)),
            pl.BlockSpec((8, H), lambda i: (0, 0)),
            pl.BlockSpec((1, H), lambda i: (0, 0)),
            pl.BlockSpec((1, H), lambda i: (0, 0)),
            pl.BlockSpec((C, H), lambda i: (0, 0)),
            pl.BlockSpec((C, 1), lambda i: (0, 0)),
            pl.BlockSpec((BR, 1), lambda i: (i, 0)),
        ],
        out_specs=[
            pl.BlockSpec((BR, 1), lambda i: (i, 0)),
            pl.BlockSpec((8, 128), lambda i: (0, 0)),
        ],
        out_shape=[
            jax.ShapeDtypeStruct((NP, 1), jnp.float32),
            jax.ShapeDtypeStruct((8, 128), jnp.float32),
        ],
    )(y, sums, g, b, wfc, bfc, seg)


def _tc_final(v, vsums, lng, lnb, n_real):
    NP = v.shape[0]

    def body(v_ref, vs_ref, g_ref, b_ref, o_ref):
        s1 = vs_ref[0:1, 0:1]
        s2 = vs_ref[1:2, 0:1]
        mu = s1 / n_real
        var = s2 / n_real - mu * mu
        inv = lax.rsqrt(var + 1e-5)
        o_ref[...] = g_ref[...] * (v_ref[...] - mu) * inv + b_ref[...]

    return pl.pallas_call(
        body,
        in_specs=[
            pl.BlockSpec((NP, 1), lambda: (0, 0)),
            pl.BlockSpec((8, 128), lambda: (0, 0)),
            pl.BlockSpec((NP, 1), lambda: (0, 0)),
            pl.BlockSpec((NP, 1), lambda: (0, 0)),
        ],
        out_specs=pl.BlockSpec((NP, 1), lambda: (0, 0)),
        out_shape=jax.ShapeDtypeStruct((NP, 1), jnp.float32),
    )(v, vsums, lng, lnb)


def kernel(features, adj, segment, W1, g1, b1, W2, g2, b2, Wf, bf, lng, lnb):
    N, F = features.shape
    E = adj.shape[1]
    C, K, _, H = W1.shape

    NP = (N // 256 + 1) * 256
    PAD_ROWS = NP - N
    EQ = LANE * LANE
    EP = ((E + EQ - 1) // EQ) * EQ
    EC = EP // LANE
    CPW = EC // NW
    RPT = NP // NS
    BR = 1024 if NP % 1024 == 0 else 512

    f32 = jnp.float32

    pad_idx = N + (jnp.arange(EP - E, dtype=jnp.int32) % PAD_ROWS)
    src2d = jnp.concatenate([adj[0], pad_idx]).reshape(EC, LANE)
    dst2d = jnp.concatenate([adj[1], pad_idx]).reshape(EC, LANE)
    idx2f = jnp.stack([src2d, dst2d], axis=1).reshape(2 * EC, LANE)
    zpad_rows = jnp.zeros((PAD_ROWS, F), f32)
    x_pad = jnp.concatenate([features.astype(f32), zpad_rows])
    seg_pad = jnp.concatenate([segment, jnp.zeros((PAD_ROWS,), jnp.int32)])
    seg_pad = seg_pad.reshape(NP, 1)
    zrows = jnp.zeros((RPT, F), f32)
    zrow1 = jnp.zeros((RPT,), f32)
    ones_row = jnp.ones((LANE,), f32)
    W1c = W1.reshape(C, K * F, H).astype(f32)
    W2c = W2.reshape(C, K * H, H).astype(f32)
    g1r = g1.reshape(1, H)
    b1r = b1.reshape(1, H)
    g2r = g2.reshape(1, H)
    b2r = b2.reshape(1, H)
    wfc = Wf[:, :, 0].astype(f32)
    lng_p = jnp.concatenate([lng, jnp.zeros((PAD_ROWS,), f32)]).reshape(NP, 1)
    lnb_p = jnp.concatenate([lnb, jnp.zeros((PAD_ROWS,), f32)]).reshape(NP, 1)
    n_real = float(N)

    degp = _sc_deg(idx2f, ones_row, zrow1, NP, CPW)
    deg0 = degp[0].reshape(NP, 1)
    deg1 = degp[1].reshape(NP, 1)

    u0, dinv = _tc_pre(deg0, deg1, x_pad, BR)

    def cheb_layer(x_in, u_in, wcat):
        p = _sc_prop(u_in, idx2f, zrows, NP, CPW)
        tx1, u1 = _tc_mid(p.reshape(2 * NP, F), dinv, BR)
        q = _sc_prop(u1, idx2f, zrows, NP, CPW)
        return _tc_layer(q.reshape(2 * NP, F), dinv, x_in, tx1, wcat,
                         seg_pad, BR)

    y1, s1 = cheb_layer(x_pad, u0, W1c)
    x1, u1b = _tc_post1(y1, s1, g1r, b1r, dinv, n_real, BR)
    y2, s2 = cheb_layer(x1, u1b, W2c)
    v, vs = _tc_post2(y2, s2, g2r, b2r, wfc, bf.astype(f32), seg_pad, n_real, BR)
    o = _tc_final(v, vs, lng_p, lnb_p, n_real)
    return o.reshape(-1)[:N]


# speedup vs baseline: 15.2501x; 1.0077x over previous
import functools

import jax
import jax.numpy as jnp
from jax import lax
from jax.experimental import pallas as pl
from jax.experimental.pallas import tpu as pltpu
from jax.experimental.pallas import tpu_sc as plsc

NC = 2
NS = 16
NW = NC * NS
LANE = 128
NBUF = 4


def _sc_mesh():
    return plsc.VectorSubcoreMesh(
        core_axis_name="c", subcore_axis_name="s", num_cores=NC, num_subcores=NS
    )


def _sc_deg(idx3, ones_row, zrow, NP, CPW):
    RPT = NP // NS

    @functools.partial(
        pl.kernel,
        out_type=jax.ShapeDtypeStruct((NC, NP), jnp.float32),
        mesh=_sc_mesh(),
        scratch_types=[
            pltpu.VMEM((NBUF, LANE), jnp.int32),
            pltpu.VMEM((LANE,), jnp.float32),
            pltpu.VMEM_SHARED((NP,), jnp.float32),
            pltpu.SemaphoreType.DMA((NBUF,)),
        ],
    )
    def k(idx_hbm, ones_hbm, z_hbm, out_hbm, dst_v, ones_v, acc, isem):
        ci = lax.axis_index("c")
        si = lax.axis_index("s")
        wid = si * NC + ci
        pltpu.sync_copy(z_hbm, acc.at[pl.ds(si * RPT, RPT)])
        pltpu.sync_copy(ones_hbm, ones_v)
        plsc.subcore_barrier()
        base = wid * CPW
        for b in range(NBUF):
            pltpu.async_copy(idx_hbm.at[(base + b) * 2 + 1], dst_v.at[b],
                             isem.at[b])

        def group(g, carry):
            j = base + g * NBUF
            for b in range(NBUF):
                pltpu.make_async_copy(
                    idx_hbm.at[(j + b) * 2 + 1], dst_v.at[b], isem.at[b]).wait()
                pltpu.sync_copy(ones_v, acc.at[dst_v.at[b]], add=True)
                pltpu.async_copy(
                    idx_hbm.at[(j + b + NBUF) * 2 + 1], dst_v.at[b], isem.at[b])
            return carry

        lax.fori_loop(0, CPW // NBUF - 1, group, 0)
        j = base + CPW - NBUF
        for b in range(NBUF):
            pltpu.make_async_copy(
                idx_hbm.at[(j + b) * 2 + 1], dst_v.at[b], isem.at[b]).wait()
            pltpu.sync_copy(ones_v, acc.at[dst_v.at[b]], add=True)
        plsc.subcore_barrier()
        pltpu.sync_copy(acc.at[pl.ds(si * RPT, RPT)],
                        out_hbm.at[ci, pl.ds(si * RPT, RPT)])

    return k(idx3, ones_row, zrow)


def _sc_prop(y, idx2, zrows, NP, CPW):
    RPT = NP // NS
    F = y.shape[1]
    NI = 4

    @functools.partial(
        pl.kernel,
        out_type=jax.ShapeDtypeStruct((NC, NP, F), jnp.float32),
        mesh=_sc_mesh(),
        scratch_types=[
            pltpu.VMEM((NI, 2, LANE), jnp.int32),
            pltpu.VMEM((2, LANE, F), jnp.float32),
            pltpu.VMEM_SHARED((NP, F), jnp.float32),
            pltpu.SemaphoreType.DMA((2,)),
            pltpu.SemaphoreType.DMA((2,)),
            pltpu.SemaphoreType.DMA((NI,)),
        ],
    )
    def k(y_hbm, idx_hbm, z_hbm, out_hbm, idx_v, rows_v, acc, gsem, ssem, isem):
        ci = lax.axis_index("c")
        si = lax.axis_index("s")
        wid = si * NC + ci
        base = wid * CPW

        def fetch_idx(ch, q):
            pltpu.async_copy(idx_hbm.at[ch * 2], idx_v.at[q, 0], isem.at[q])
            pltpu.async_copy(idx_hbm.at[ch * 2 + 1], idx_v.at[q, 1], isem.at[q])

        def wait_idx(q):
            for r in range(2):
                pltpu.make_async_copy(
                    idx_hbm.at[0], idx_v.at[q, r], isem.at[q]).wait()

        def issue_gather(q, b):
            pltpu.async_copy(y_hbm.at[idx_v.at[q, 0]], rows_v.at[b],
                             gsem.at[b])

        def wait_gather(q, b):
            pltpu.make_async_copy(y_hbm.at[idx_v.at[q, 0]], rows_v.at[b],
                                  gsem.at[b]).wait()

        def issue_scatter(q, b):
            pltpu.async_copy(rows_v.at[b], acc.at[idx_v.at[q, 1]], ssem.at[b],
                             add=True)

        def wait_scatter(q, b):
            pltpu.make_async_copy(rows_v.at[b], acc.at[idx_v.at[q, 1]],
                                  ssem.at[b]).wait()

        def step(j, jm2, jm4, do_c, do_d, do_ef):
            b, b2 = jm2, 1 - jm2
            q1, q3 = (jm4 + 1) % NI, (jm4 + 3) % NI
            wait_gather(jm4, b)
            issue_scatter(jm4, b)
            if do_c:
                wait_scatter(q3, b2)
            if do_d:
                fetch_idx(j + 3, q3)
            if do_ef:
                wait_idx(q1)
                issue_gather(q1, b2)

        fetch_idx(base, 0)
        fetch_idx(base + 1, 1)
        fetch_idx(base + 2, 2)
        wait_idx(0)
        issue_gather(0, 0)
        pltpu.sync_copy(z_hbm, acc.at[pl.ds(si * RPT, RPT)])
        plsc.subcore_barrier()
        step(base + 0, 0, 0, False, True, True)
        step(base + 1, 1, 1, True, True, True)
        step(base + 2, 0, 2, True, True, True)
        step(base + 3, 1, 3, True, True, True)

        def group(g, carry):
            j = base + 4 + g * 4
            for t in range(4):
                step(j + t, t % 2, t, True, True, True)
            return carry

        lax.fori_loop(0, (CPW - 8) // 4, group, 0)
        jt = base + CPW - 4
        step(jt + 0, 0, 0, True, True, True)
        step(jt + 1, 1, 1, True, False, True)
        step(jt + 2, 0, 2, True, False, True)
        step(jt + 3, 1, 3, True, False, False)
        wait_scatter(3, 1)
        plsc.subcore_barrier()
        pltpu.sync_copy(acc.at[pl.ds(si * RPT, RPT)],
                        out_hbm.at[ci, pl.ds(si * RPT, RPT)])

    return k(y, idx2, zrows)


def _tc_pre(deg0, deg1, x, BR):
    NP, F = x.shape
    G = NP // BR

    def body(d0_ref, d1_ref, x_ref, u_ref, dinv_ref):
        deg = jnp.maximum(d0_ref[...] + d1_ref[...], 1.0)
        dinv = lax.rsqrt(deg)
        dinv_ref[...] = dinv
        u_ref[...] = x_ref[...] * dinv

    return pl.pallas_call(
        body,
        grid=(G,),
        in_specs=[
            pl.BlockSpec((BR, 1), lambda i: (i, 0)),
            pl.BlockSpec((BR, 1), lambda i: (i, 0)),
            pl.BlockSpec((BR, F), lambda i: (i, 0)),
        ],
        out_specs=[
            pl.BlockSpec((BR, F), lambda i: (i, 0)),
            pl.BlockSpec((BR, 1), lambda i: (i, 0)),
        ],
        out_shape=[
            jax.ShapeDtypeStruct((NP, F), jnp.float32),
            jax.ShapeDtypeStruct((NP, 1), jnp.float32),
        ],
    )(deg0, deg1, x)


def _tc_mid(p2, dinv, BR):
    NP2, F = p2.shape
    NP = NP2 // 2
    G = NP // BR

    def body(pa_ref, pb_ref, dv_ref, tx1_ref, u_ref):
        dv = dv_ref[...]
        tx1 = -dv * (pa_ref[...] + pb_ref[...])
        tx1_ref[...] = tx1
        u_ref[...] = dv * tx1

    return pl.pallas_call(
        body,
        grid=(G,),
        in_specs=[
            pl.BlockSpec((BR, F), lambda i: (i, 0)),
            pl.BlockSpec((BR, F), lambda i: (G + i, 0)),
            pl.BlockSpec((BR, 1), lambda i: (i, 0)),
        ],
        out_specs=[
            pl.BlockSpec((BR, F), lambda i: (i, 0)),
            pl.BlockSpec((BR, F), lambda i: (i, 0)),
        ],
        out_shape=[
            jax.ShapeDtypeStruct((NP, F), jnp.float32),
            jax.ShapeDtypeStruct((NP, F), jnp.float32),
        ],
    )(p2, p2, dinv)


def _tc_layer(q2, dinv, x_in, tx1, wcat, seg, BR):
    NP, F = x_in.shape
    C, D3, H = wcat.shape
    G = NP // BR

    def body(qa_ref, qb_ref, dv_ref, x_ref, t1_ref, w_ref, seg_ref, y_ref, s_ref):
        i = pl.program_id(0)
        dv = dv_ref[...]
        x = x_ref[...]
        t1 = t1_ref[...]
        q = qa_ref[...] + qb_ref[...]
        t2 = -2.0 * dv * q - x
        t = jnp.concatenate([x, t1, t2], axis=1)
        seg = seg_ref[...]
        tb = t.astype(jnp.bfloat16)
        y = jnp.zeros((BR, H), jnp.float32)
        for c in range(C):
            m = jnp.where(seg == c, 1.0, 0.0)
            y = y + m * jnp.dot(tb, w_ref[c],
                                preferred_element_type=jnp.float32)
        y_ref[...] = y
        s1 = jnp.sum(y, axis=0, keepdims=True)
        s2 = jnp.sum(y * y, axis=0, keepdims=True)
        rows = lax.broadcasted_iota(jnp.int32, (8, H), 0)
        sblk = (jnp.where(rows == 0, jnp.broadcast_to(s1, (8, H)), 0.0)
                + jnp.where(rows == 1, jnp.broadcast_to(s2, (8, H)), 0.0))

        @pl.when(i == 0)
        def _():
            s_ref[...] = sblk

        @pl.when(i > 0)
        def _():
            s_ref[...] = s_ref[...] + sblk

    return pl.pallas_call(
        body,
        grid=(G,),
        in_specs=[
            pl.BlockSpec((BR, F), lambda i: (i, 0)),
            pl.BlockSpec((BR, F), lambda i: (G + i, 0)),
            pl.BlockSpec((BR, 1), lambda i: (i, 0)),
            pl.BlockSpec((BR, F), lambda i: (i, 0)),
            pl.BlockSpec((BR, F), lambda i: (i, 0)),
            pl.BlockSpec((C, D3, H), lambda i: (0, 0, 0)),
            pl.BlockSpec((BR, 1), lambda i: (i, 0)),
        ],
        out_specs=[
            pl.BlockSpec((BR, H), lambda i: (i, 0)),
            pl.BlockSpec((8, H), lambda i: (0, 0)),
        ],
        out_shape=[
            jax.ShapeDtypeStruct((NP, H), jnp.float32),
            jax.ShapeDtypeStruct((8, H), jnp.float32),
        ],
    )(q2, q2, dinv, x_in, tx1, wcat, seg)


def _tc_post1(y, sums, g, b, dinv, n_real, BR):
    NP, H = y.shape
    G = NP // BR

    def body(y_ref, s_ref, g_ref, b_ref, dv_ref, x_ref, u_ref):
        i = pl.program_id(0)
        s1 = s_ref[0:1, :]
        s2 = s_ref[1:2, :]
        mu = s1 / n_real
        var = s2 / n_real - mu * mu
        inv = lax.rsqrt(var + 1e-5)
        yb = g_ref[...] * (y_ref[...] - mu) * inv + b_ref[...]
        xn = yb * (1.0 / (1.0 + jnp.exp(-yb)))
        rows = i * BR + lax.broadcasted_iota(jnp.int32, (BR, 1), 0)
        xn = jnp.where(rows < n_real, xn, 0.0)
        x_ref[...] = xn
        u_ref[...] = dv_ref[...] * xn

    return pl.pallas_call(
        body,
        grid=(G,),
        in_specs=[
            pl.BlockSpec((BR, H), lambda i: (i, 0)),
            pl.BlockSpec((8, H), lambda i: (0, 0)),
            pl.BlockSpec((1, H), lambda i: (0, 0)),
            pl.BlockSpec((1, H), lambda i: (0, 0)),
            pl.BlockSpec((BR, 1), lambda i: (i, 0)),
        ],
        out_specs=[
            pl.BlockSpec((BR, H), lambda i: (i, 0)),
            pl.BlockSpec((BR, H), lambda i: (i, 0)),
        ],
        out_shape=[
            jax.ShapeDtypeStruct((NP, H), jnp.float32),
            jax.ShapeDtypeStruct((NP, H), jnp.float32),
        ],
    )(y, sums, g, b, dinv)


def _tc_post2f(y, sums, g, b, wfc, bfc, seg, lng, lnb, n_real):
    NP, H = y.shape
    C = wfc.shape[0]

    def body(y_ref, s_ref, g_ref, b_ref, wf_ref, bf_ref, seg_ref, lg_ref,
             lb_ref, o_ref):
        s1 = s_ref[0:1, :]
        s2 = s_ref[1:2, :]
        mu = s1 / n_real
        var = s2 / n_real - mu * mu
        inv = lax.rsqrt(var + 1e-5)
        yb = g_ref[...] * (y_ref[...] - mu) * inv + b_ref[...]
        xn = jnp.tanh(yb)
        seg = seg_ref[...]
        wsel = jnp.zeros((NP, H), jnp.float32)
        bsel = jnp.zeros((NP, 1), jnp.float32)
        for c in range(C):
            m = jnp.where(seg == c, 1.0, 0.0)
            wsel = wsel + m * wf_ref[c:c + 1, :]
            bsel = bsel + m * bf_ref[c, 0]
        v = jnp.sum(xn * wsel, axis=1, keepdims=True) + bsel
        rows = lax.broadcasted_iota(jnp.int32, (NP, 1), 0)
        v = jnp.where(rows < n_real, v, 0.0)
        sv1 = jnp.sum(v)
        sv2 = jnp.sum(v * v)
        vmu = sv1 / n_real
        vvar = sv2 / n_real - vmu * vmu
        vinv = lax.rsqrt(vvar + 1e-5)
        o_ref[...] = lg_ref[...] * (v - vmu) * vinv + lb_ref[...]

    return pl.pallas_call(
        body,
        in_specs=[
            pl.BlockSpec((NP, H), lambda: (0, 0)),
            pl.BlockSpec((8, H), lambda: (0, 0)),
            pl.BlockSpec((1, H), lambda: (0, 0)),
            pl.BlockSpec((1, H), lambda: (0, 0)),
            pl.BlockSpec((C, H), lambda: (0, 0)),
            pl.BlockSpec((C, 1), lambda: (0, 0)),
            pl.BlockSpec((NP, 1), lambda: (0, 0)),
            pl.BlockSpec((NP, 1), lambda: (0, 0)),
            pl.BlockSpec((NP, 1), lambda: (0, 0)),
        ],
        out_specs=pl.BlockSpec((NP, 1), lambda: (0, 0)),
        out_shape=jax.ShapeDtypeStruct((NP, 1), jnp.float32),
    )(y, sums, g, b, wfc, bfc, seg, lng, lnb)


def kernel(features, adj, segment, W1, g1, b1, W2, g2, b2, Wf, bf, lng, lnb):
    N, F = features.shape
    E = adj.shape[1]
    C, K, _, H = W1.shape

    NP = (N // 256 + 1) * 256
    PAD_ROWS = NP - N
    EQ = LANE * LANE
    EP = ((E + EQ - 1) // EQ) * EQ
    EC = EP // LANE
    CPW = EC // NW
    RPT = NP // NS
    BR = 1024 if NP % 1024 == 0 else 512

    f32 = jnp.float32

    pad_idx = N + (jnp.arange(EP - E, dtype=jnp.int32) % PAD_ROWS)
    src2d = jnp.concatenate([adj[0], pad_idx]).reshape(EC, LANE)
    dst2d = jnp.concatenate([adj[1], pad_idx]).reshape(EC, LANE)
    idx2f = jnp.stack([src2d, dst2d], axis=1).reshape(2 * EC, LANE)
    zpad_rows = jnp.zeros((PAD_ROWS, F), f32)
    x_pad = jnp.concatenate([features.astype(f32), zpad_rows])
    seg_pad = jnp.concatenate([segment, jnp.zeros((PAD_ROWS,), jnp.int32)])
    seg_pad = seg_pad.reshape(NP, 1)
    zrows = jnp.zeros((RPT, F), f32)
    zrow1 = jnp.zeros((RPT,), f32)
    ones_row = jnp.ones((LANE,), f32)
    W1c = W1.reshape(C, K * F, H).astype(jnp.bfloat16)
    W2c = W2.reshape(C, K * H, H).astype(jnp.bfloat16)
    g1r = g1.reshape(1, H)
    b1r = b1.reshape(1, H)
    g2r = g2.reshape(1, H)
    b2r = b2.reshape(1, H)
    wfc = Wf[:, :, 0].astype(f32)
    lng_p = jnp.concatenate([lng, jnp.zeros((PAD_ROWS,), f32)]).reshape(NP, 1)
    lnb_p = jnp.concatenate([lnb, jnp.zeros((PAD_ROWS,), f32)]).reshape(NP, 1)
    n_real = float(N)

    degp = _sc_deg(idx2f, ones_row, zrow1, NP, CPW)
    deg0 = degp[0].reshape(NP, 1)
    deg1 = degp[1].reshape(NP, 1)

    u0, dinv = _tc_pre(deg0, deg1, x_pad, BR)

    def cheb_layer(x_in, u_in, wcat):
        p = _sc_prop(u_in, idx2f, zrows, NP, CPW)
        tx1, u1 = _tc_mid(p.reshape(2 * NP, F), dinv, BR)
        q = _sc_prop(u1, idx2f, zrows, NP, CPW)
        return _tc_layer(q.reshape(2 * NP, F), dinv, x_in, tx1, wcat,
                         seg_pad, BR)

    y1, s1 = cheb_layer(x_pad, u0, W1c)
    x1, u1b = _tc_post1(y1, s1, g1r, b1r, dinv, n_real, BR)
    y2, s2 = cheb_layer(x1, u1b, W2c)
    o = _tc_post2f(y2, s2, g2r, b2r, wfc, bf.astype(f32), seg_pad,
                   lng_p, lnb_p, n_real)
    return o.reshape(-1)[:N]


# speedup vs baseline: 15.2967x; 1.0031x over previous
import functools

import jax
import jax.numpy as jnp
from jax import lax
from jax.experimental import pallas as pl
from jax.experimental.pallas import tpu as pltpu
from jax.experimental.pallas import tpu_sc as plsc

NC = 2
NS = 16
NW = NC * NS
LANE = 128
NBUF = 4


def _sc_mesh():
    return plsc.VectorSubcoreMesh(
        core_axis_name="c", subcore_axis_name="s", num_cores=NC, num_subcores=NS
    )


def _sc_deg(idx3, ones_row, zrow, NP, CPW):
    RPT = NP // NS

    @functools.partial(
        pl.kernel,
        out_type=jax.ShapeDtypeStruct((NC, NP), jnp.float32),
        mesh=_sc_mesh(),
        scratch_types=[
            pltpu.VMEM((NBUF, LANE), jnp.int32),
            pltpu.VMEM((LANE,), jnp.float32),
            pltpu.VMEM_SHARED((NP,), jnp.float32),
            pltpu.SemaphoreType.DMA((NBUF,)),
        ],
    )
    def k(idx_hbm, ones_hbm, z_hbm, out_hbm, dst_v, ones_v, acc, isem):
        ci = lax.axis_index("c")
        si = lax.axis_index("s")
        wid = si * NC + ci
        pltpu.sync_copy(z_hbm, acc.at[pl.ds(si * RPT, RPT)])
        pltpu.sync_copy(ones_hbm, ones_v)
        plsc.subcore_barrier()
        base = wid * CPW
        for b in range(NBUF):
            pltpu.async_copy(idx_hbm.at[(base + b) * 2 + 1], dst_v.at[b],
                             isem.at[b])

        def group(g, carry):
            j = base + g * NBUF
            for b in range(NBUF):
                pltpu.make_async_copy(
                    idx_hbm.at[(j + b) * 2 + 1], dst_v.at[b], isem.at[b]).wait()
                pltpu.sync_copy(ones_v, acc.at[dst_v.at[b]], add=True)
                pltpu.async_copy(
                    idx_hbm.at[(j + b + NBUF) * 2 + 1], dst_v.at[b], isem.at[b])
            return carry

        lax.fori_loop(0, CPW // NBUF - 1, group, 0)
        j = base + CPW - NBUF
        for b in range(NBUF):
            pltpu.make_async_copy(
                idx_hbm.at[(j + b) * 2 + 1], dst_v.at[b], isem.at[b]).wait()
            pltpu.sync_copy(ones_v, acc.at[dst_v.at[b]], add=True)
        plsc.subcore_barrier()
        pltpu.sync_copy(acc.at[pl.ds(si * RPT, RPT)],
                        out_hbm.at[ci, pl.ds(si * RPT, RPT)])

    return k(idx3, ones_row, zrow)


def _sc_prop(y, idx2, zrows, NP, CPW):
    RPT = NP // NS
    F = y.shape[1]
    GPW = CPW // 4

    @functools.partial(
        pl.kernel,
        out_type=jax.ShapeDtypeStruct((NC, NP, F), jnp.float32),
        mesh=_sc_mesh(),
        scratch_types=[
            pltpu.VMEM((2, 8, LANE), jnp.int32),
            pltpu.VMEM((2, LANE, F), jnp.float32),
            pltpu.VMEM_SHARED((NP, F), jnp.float32),
            pltpu.SemaphoreType.DMA((2,)),
            pltpu.SemaphoreType.DMA((2,)),
            pltpu.SemaphoreType.DMA((2,)),
        ],
    )
    def k(y_hbm, idx_hbm, z_hbm, out_hbm, idxg, rows_v, acc, gsem, ssem, isem):
        ci = lax.axis_index("c")
        si = lax.axis_index("s")
        wid = si * NC + ci
        base_row = wid * CPW * 2

        def fetch_group(g, gb):
            row = pl.multiple_of(base_row + g * 8, 8)
            pltpu.async_copy(idx_hbm.at[pl.ds(row, 8)], idxg.at[gb],
                             isem.at[gb])

        def wait_group(gb):
            pltpu.make_async_copy(idx_hbm.at[pl.ds(0, 8)], idxg.at[gb],
                                  isem.at[gb]).wait()

        def issue_gather(gb, t, b):
            pltpu.async_copy(y_hbm.at[idxg.at[gb, 2 * t]], rows_v.at[b],
                             gsem.at[b])

        def wait_gather(gb, t, b):
            pltpu.make_async_copy(y_hbm.at[idxg.at[gb, 2 * t]], rows_v.at[b],
                                  gsem.at[b]).wait()

        def issue_scatter(gb, t, b):
            pltpu.async_copy(rows_v.at[b], acc.at[idxg.at[gb, 2 * t + 1]],
                             ssem.at[b], add=True)

        def wait_scatter(gb, t, b):
            pltpu.make_async_copy(rows_v.at[b], acc.at[idxg.at[gb, 2 * t + 1]],
                                  ssem.at[b]).wait()

        def group_body(g, gb, first, last):
            for t in range(4):
                b = t % 2
                b2 = 1 - b
                wait_gather(gb, t, b)
                issue_scatter(gb, t, b)
                if t == 0:
                    if not first:
                        wait_scatter(1 - gb, 3, b2)
                    if not last:
                        fetch_group(g + 1, 1 - gb)
                else:
                    wait_scatter(gb, t - 1, b2)
                if t < 3:
                    issue_gather(gb, t + 1, b2)
                elif not last:
                    wait_group(1 - gb)
                    issue_gather(1 - gb, 0, b2)

        fetch_group(0, 0)
        wait_group(0)
        issue_gather(0, 0, 0)
        pltpu.sync_copy(z_hbm, acc.at[pl.ds(si * RPT, RPT)])
        plsc.subcore_barrier()
        group_body(0, 0, True, False)

        def loop(g2, carry):
            g = 1 + 2 * g2
            group_body(g, 1, False, False)
            group_body(g + 1, 0, False, False)
            return carry

        lax.fori_loop(0, (GPW - 2) // 2, loop, 0)
        group_body(GPW - 1, (GPW - 1) % 2, False, True)
        wait_scatter((GPW - 1) % 2, 3, 1)
        plsc.subcore_barrier()
        pltpu.sync_copy(acc.at[pl.ds(si * RPT, RPT)],
                        out_hbm.at[ci, pl.ds(si * RPT, RPT)])

    return k(y, idx2, zrows)


def _tc_pre(deg0, deg1, x, BR):
    NP, F = x.shape
    G = NP // BR

    def body(d0_ref, d1_ref, x_ref, u_ref, dinv_ref):
        deg = jnp.maximum(d0_ref[...] + d1_ref[...], 1.0)
        dinv = lax.rsqrt(deg)
        dinv_ref[...] = dinv
        u_ref[...] = x_ref[...] * dinv

    return pl.pallas_call(
        body,
        grid=(G,),
        in_specs=[
            pl.BlockSpec((BR, 1), lambda i: (i, 0)),
            pl.BlockSpec((BR, 1), lambda i: (i, 0)),
            pl.BlockSpec((BR, F), lambda i: (i, 0)),
        ],
        out_specs=[
            pl.BlockSpec((BR, F), lambda i: (i, 0)),
            pl.BlockSpec((BR, 1), lambda i: (i, 0)),
        ],
        out_shape=[
            jax.ShapeDtypeStruct((NP, F), jnp.float32),
            jax.ShapeDtypeStruct((NP, 1), jnp.float32),
        ],
    )(deg0, deg1, x)


def _tc_mid(p2, dinv, BR):
    NP2, F = p2.shape
    NP = NP2 // 2
    G = NP // BR

    def body(pa_ref, pb_ref, dv_ref, tx1_ref, u_ref):
        dv = dv_ref[...]
        tx1 = -dv * (pa_ref[...] + pb_ref[...])
        tx1_ref[...] = tx1
        u_ref[...] = dv * tx1

    return pl.pallas_call(
        body,
        grid=(G,),
        in_specs=[
            pl.BlockSpec((BR, F), lambda i: (i, 0)),
            pl.BlockSpec((BR, F), lambda i: (G + i, 0)),
            pl.BlockSpec((BR, 1), lambda i: (i, 0)),
        ],
        out_specs=[
            pl.BlockSpec((BR, F), lambda i: (i, 0)),
            pl.BlockSpec((BR, F), lambda i: (i, 0)),
        ],
        out_shape=[
            jax.ShapeDtypeStruct((NP, F), jnp.float32),
            jax.ShapeDtypeStruct((NP, F), jnp.float32),
        ],
    )(p2, p2, dinv)


def _tc_layer(q2, dinv, x_in, tx1, wcat, seg, BR):
    NP, F = x_in.shape
    C, D3, H = wcat.shape
    G = NP // BR

    def body(qa_ref, qb_ref, dv_ref, x_ref, t1_ref, w_ref, seg_ref, y_ref, s_ref):
        i = pl.program_id(0)
        dv = dv_ref[...]
        x = x_ref[...]
        t1 = t1_ref[...]
        q = qa_ref[...] + qb_ref[...]
        t2 = -2.0 * dv * q - x
        t = jnp.concatenate([x, t1, t2], axis=1)
        seg = seg_ref[...]
        tb = t.astype(jnp.bfloat16)
        y = jnp.zeros((BR, H), jnp.float32)
        for c in range(C):
            m = jnp.where(seg == c, 1.0, 0.0)
            y = y + m * jnp.dot(tb, w_ref[c],
                                preferred_element_type=jnp.float32)
        y_ref[...] = y
        s1 = jnp.sum(y, axis=0, keepdims=True)
        s2 = jnp.sum(y * y, axis=0, keepdims=True)
        rows = lax.broadcasted_iota(jnp.int32, (8, H), 0)
        sblk = (jnp.where(rows == 0, jnp.broadcast_to(s1, (8, H)), 0.0)
                + jnp.where(rows == 1, jnp.broadcast_to(s2, (8, H)), 0.0))

        @pl.when(i == 0)
        def _():
            s_ref[...] = sblk

        @pl.when(i > 0)
        def _():
            s_ref[...] = s_ref[...] + sblk

    return pl.pallas_call(
        body,
        grid=(G,),
        in_specs=[
            pl.BlockSpec((BR, F), lambda i: (i, 0)),
            pl.BlockSpec((BR, F), lambda i: (G + i, 0)),
            pl.BlockSpec((BR, 1), lambda i: (i, 0)),
            pl.BlockSpec((BR, F), lambda i: (i, 0)),
            pl.BlockSpec((BR, F), lambda i: (i, 0)),
            pl.BlockSpec((C, D3, H), lambda i: (0, 0, 0)),
            pl.BlockSpec((BR, 1), lambda i: (i, 0)),
        ],
        out_specs=[
            pl.BlockSpec((BR, H), lambda i: (i, 0)),
            pl.BlockSpec((8, H), lambda i: (0, 0)),
        ],
        out_shape=[
            jax.ShapeDtypeStruct((NP, H), jnp.float32),
            jax.ShapeDtypeStruct((8, H), jnp.float32),
        ],
    )(q2, q2, dinv, x_in, tx1, wcat, seg)


def _tc_post1(y, sums, g, b, dinv, n_real, BR):
    NP, H = y.shape
    G = NP // BR

    def body(y_ref, s_ref, g_ref, b_ref, dv_ref, x_ref, u_ref):
        i = pl.program_id(0)
        s1 = s_ref[0:1, :]
        s2 = s_ref[1:2, :]
        mu = s1 / n_real
        var = s2 / n_real - mu * mu
        inv = lax.rsqrt(var + 1e-5)
        yb = g_ref[...] * (y_ref[...] - mu) * inv + b_ref[...]
        xn = yb * (1.0 / (1.0 + jnp.exp(-yb)))
        rows = i * BR + lax.broadcasted_iota(jnp.int32, (BR, 1), 0)
        xn = jnp.where(rows < n_real, xn, 0.0)
        x_ref[...] = xn
        u_ref[...] = dv_ref[...] * xn

    return pl.pallas_call(
        body,
        grid=(G,),
        in_specs=[
            pl.BlockSpec((BR, H), lambda i: (i, 0)),
            pl.BlockSpec((8, H), lambda i: (0, 0)),
            pl.BlockSpec((1, H), lambda i: (0, 0)),
            pl.BlockSpec((1, H), lambda i: (0, 0)),
            pl.BlockSpec((BR, 1), lambda i: (i, 0)),
        ],
        out_specs=[
            pl.BlockSpec((BR, H), lambda i: (i, 0)),
            pl.BlockSpec((BR, H), lambda i: (i, 0)),
        ],
        out_shape=[
            jax.ShapeDtypeStruct((NP, H), jnp.float32),
            jax.ShapeDtypeStruct((NP, H), jnp.float32),
        ],
    )(y, sums, g, b, dinv)


def _tc_post2f(y, sums, g, b, wfc, bfc, seg, lng, lnb, n_real):
    NP, H = y.shape
    C = wfc.shape[0]

    def body(y_ref, s_ref, g_ref, b_ref, wf_ref, bf_ref, seg_ref, lg_ref,
             lb_ref, o_ref):
        s1 = s_ref[0:1, :]
        s2 = s_ref[1:2, :]
        mu = s1 / n_real
        var = s2 / n_real - mu * mu
        inv = lax.rsqrt(var + 1e-5)
        yb = g_ref[...] * (y_ref[...] - mu) * inv + b_ref[...]
        xn = jnp.tanh(yb)
        seg = seg_ref[...]
        wsel = jnp.zeros((NP, H), jnp.float32)
        bsel = jnp.zeros((NP, 1), jnp.float32)
        for c in range(C):
            m = jnp.where(seg == c, 1.0, 0.0)
            wsel = wsel + m * wf_ref[c:c + 1, :]
            bsel = bsel + m * bf_ref[c, 0]
        v = jnp.sum(xn * wsel, axis=1, keepdims=True) + bsel
        rows = lax.broadcasted_iota(jnp.int32, (NP, 1), 0)
        v = jnp.where(rows < n_real, v, 0.0)
        sv1 = jnp.sum(v)
        sv2 = jnp.sum(v * v)
        vmu = sv1 / n_real
        vvar = sv2 / n_real - vmu * vmu
        vinv = lax.rsqrt(vvar + 1e-5)
        o_ref[...] = lg_ref[...] * (v - vmu) * vinv + lb_ref[...]

    return pl.pallas_call(
        body,
        in_specs=[
            pl.BlockSpec((NP, H), lambda: (0, 0)),
            pl.BlockSpec((8, H), lambda: (0, 0)),
            pl.BlockSpec((1, H), lambda: (0, 0)),
            pl.BlockSpec((1, H), lambda: (0, 0)),
            pl.BlockSpec((C, H), lambda: (0, 0)),
            pl.BlockSpec((C, 1), lambda: (0, 0)),
            pl.BlockSpec((NP, 1), lambda: (0, 0)),
            pl.BlockSpec((NP, 1), lambda: (0, 0)),
            pl.BlockSpec((NP, 1), lambda: (0, 0)),
        ],
        out_specs=pl.BlockSpec((NP, 1), lambda: (0, 0)),
        out_shape=jax.ShapeDtypeStruct((NP, 1), jnp.float32),
    )(y, sums, g, b, wfc, bfc, seg, lng, lnb)


def kernel(features, adj, segment, W1, g1, b1, W2, g2, b2, Wf, bf, lng, lnb):
    N, F = features.shape
    E = adj.shape[1]
    C, K, _, H = W1.shape

    NP = (N // 256 + 1) * 256
    PAD_ROWS = NP - N
    EQ = LANE * LANE
    EP = ((E + EQ - 1) // EQ) * EQ
    EC = EP // LANE
    CPW = EC // NW
    RPT = NP // NS
    BR = 1024 if NP % 1024 == 0 else 512

    f32 = jnp.float32

    pad_idx = N + (jnp.arange(EP - E, dtype=jnp.int32) % PAD_ROWS)
    src2d = jnp.concatenate([adj[0], pad_idx]).reshape(EC, LANE)
    dst2d = jnp.concatenate([adj[1], pad_idx]).reshape(EC, LANE)
    idx2f = jnp.stack([src2d, dst2d], axis=1).reshape(2 * EC, LANE)
    zpad_rows = jnp.zeros((PAD_ROWS, F), f32)
    x_pad = jnp.concatenate([features.astype(f32), zpad_rows])
    seg_pad = jnp.concatenate([segment, jnp.zeros((PAD_ROWS,), jnp.int32)])
    seg_pad = seg_pad.reshape(NP, 1)
    zrows = jnp.zeros((RPT, F), f32)
    zrow1 = jnp.zeros((RPT,), f32)
    ones_row = jnp.ones((LANE,), f32)
    W1c = W1.reshape(C, K * F, H).astype(jnp.bfloat16)
    W2c = W2.reshape(C, K * H, H).astype(jnp.bfloat16)
    g1r = g1.reshape(1, H)
    b1r = b1.reshape(1, H)
    g2r = g2.reshape(1, H)
    b2r = b2.reshape(1, H)
    wfc = Wf[:, :, 0].astype(f32)
    lng_p = jnp.concatenate([lng, jnp.zeros((PAD_ROWS,), f32)]).reshape(NP, 1)
    lnb_p = jnp.concatenate([lnb, jnp.zeros((PAD_ROWS,), f32)]).reshape(NP, 1)
    n_real = float(N)

    degp = _sc_deg(idx2f, ones_row, zrow1, NP, CPW)
    deg0 = degp[0].reshape(NP, 1)
    deg1 = degp[1].reshape(NP, 1)

    u0, dinv = _tc_pre(deg0, deg1, x_pad, BR)

    def cheb_layer(x_in, u_in, wcat):
        p = _sc_prop(u_in, idx2f, zrows, NP, CPW)
        tx1, u1 = _tc_mid(p.reshape(2 * NP, F), dinv, BR)
        q = _sc_prop(u1, idx2f, zrows, NP, CPW)
        return _tc_layer(q.reshape(2 * NP, F), dinv, x_in, tx1, wcat,
                         seg_pad, BR)

    y1, s1 = cheb_layer(x_pad, u0, W1c)
    x1, u1b = _tc_post1(y1, s1, g1r, b1r, dinv, n_real, BR)
    y2, s2 = cheb_layer(x1, u1b, W2c)
    o = _tc_post2f(y2, s2, g2r, b2r, wfc, bf.astype(f32), seg_pad,
                   lng_p, lnb_p, n_real)
    return o.reshape(-1)[:N]
